# baseline (speedup 1.0000x reference)
import functools

import jax
import jax.numpy as jnp
from jax.experimental import pallas as pl
from jax.experimental.pallas import tpu as pltpu


def _pool_kernel(x_ref, p_ref, o_ref, *, hb):
    c = x_ref.shape[2]
    w2 = o_ref.shape[3]
    x = x_ref[0].reshape(hb, 2, c, 2 * w2)
    s = x[:, 0] + x[:, 1]
    s2 = s.reshape(hb * c, 2 * w2)
    y = jax.lax.dot(s2, p_ref[...], precision=jax.lax.Precision.HIGHEST,
                    preferred_element_type=jnp.float32)
    o_ref[0] = y.reshape(hb, c, w2)


def kernel(inputs):
    b, h, w, c = inputs.shape
    h2, w2 = h // 2, w // 2

    xt = inputs.transpose(0, 1, 3, 2)
    pair = jnp.repeat(jnp.eye(w2, dtype=inputs.dtype) * 0.5, 2, axis=0)

    hb = 64
    while h2 % hb:
        hb //= 2

    out = pl.pallas_call(
        functools.partial(_pool_kernel, hb=hb),
        grid=(b, h2 // hb),
        in_specs=[
            pl.BlockSpec((1, 2 * hb, c, w), lambda bi, hi: (bi, hi, 0, 0)),
            pl.BlockSpec((w, w2), lambda bi, hi: (0, 0)),
        ],
        out_specs=pl.BlockSpec((1, hb, c, w2), lambda bi, hi: (bi, hi, 0, 0)),
        out_shape=jax.ShapeDtypeStruct((b, h2, c, w2), inputs.dtype),
        compiler_params=pltpu.CompilerParams(
            dimension_semantics=(pltpu.PARALLEL, pltpu.ARBITRARY),
        ),
    )(xt, pair)
    return out.transpose(0, 1, 3, 2)


# speedup vs baseline: 1.1417x; 1.1417x over previous
import functools

import jax
import jax.numpy as jnp
from jax.experimental import pallas as pl
from jax.experimental.pallas import tpu as pltpu


def _pool_kernel(x_ref, o_ref, *, hb):
    c = x_ref.shape[2]
    w = x_ref.shape[3]
    x = x_ref[0].reshape(hb, 2, c, w)
    s = x[:, 0] + x[:, 1]
    half = jnp.arange(64, dtype=jnp.int32)
    idx = jnp.broadcast_to(
        jnp.concatenate([2 * half, 2 * half + 1])[None, None, :], (hb, c, 128)
    )
    parts = []
    for k in range(w // 128):
        t = jnp.take_along_axis(s[..., k * 128:(k + 1) * 128], idx, axis=-1)
        parts.append(t[..., :64] + t[..., 64:])
    o_ref[0] = jnp.concatenate(parts, axis=-1) * 0.5


def kernel(inputs):
    b, h, w, c = inputs.shape
    h2, w2 = h // 2, w // 2

    xt = inputs.transpose(0, 1, 3, 2)

    hb = 64
    while h2 % hb:
        hb //= 2

    out = pl.pallas_call(
        functools.partial(_pool_kernel, hb=hb),
        grid=(b, h2 // hb),
        in_specs=[
            pl.BlockSpec((1, 2 * hb, c, w), lambda bi, hi: (bi, hi, 0, 0)),
        ],
        out_specs=pl.BlockSpec((1, hb, c, w2), lambda bi, hi: (bi, hi, 0, 0)),
        out_shape=jax.ShapeDtypeStruct((b, h2, c, w2), inputs.dtype),
        compiler_params=pltpu.CompilerParams(
            dimension_semantics=(pltpu.PARALLEL, pltpu.ARBITRARY),
        ),
    )(xt)
    return out.transpose(0, 1, 3, 2)


# speedup vs baseline: 1.1423x; 1.0005x over previous
import functools

import jax
import jax.numpy as jnp
from jax.experimental import pallas as pl
from jax.experimental.pallas import tpu as pltpu


def _pool_kernel(x_ref, o_ref, *, hb):
    c = x_ref.shape[2]
    w = x_ref.shape[3]
    x = x_ref[0].reshape(hb, 2, c, w)
    s = x[:, 0] + x[:, 1]
    half = jnp.arange(64, dtype=jnp.int32)
    idx = jnp.broadcast_to(
        jnp.concatenate([2 * half, 2 * half + 1])[None, None, :], (hb, c, 128)
    )
    for k in range(w // 128):
        t = jnp.take_along_axis(s[..., k * 128:(k + 1) * 128], idx, axis=-1)
        o_ref[0, :, :, k * 64:(k + 1) * 64] = (t[..., :64] + t[..., 64:]) * 0.5


def kernel(inputs):
    b, h, w, c = inputs.shape
    h2, w2 = h // 2, w // 2

    xt = inputs.transpose(0, 1, 3, 2)

    hb = 64
    while h2 % hb:
        hb //= 2

    out = pl.pallas_call(
        functools.partial(_pool_kernel, hb=hb),
        grid=(b, h2 // hb),
        in_specs=[
            pl.BlockSpec((1, 2 * hb, c, w), lambda bi, hi: (bi, hi, 0, 0)),
        ],
        out_specs=pl.BlockSpec((1, hb, c, w2), lambda bi, hi: (bi, hi, 0, 0)),
        out_shape=jax.ShapeDtypeStruct((b, h2, c, w2), inputs.dtype),
        compiler_params=pltpu.CompilerParams(
            dimension_semantics=(pltpu.PARALLEL, pltpu.ARBITRARY),
        ),
    )(xt)
    return out.transpose(0, 1, 3, 2)


# speedup vs baseline: 1.3083x; 1.1453x over previous
import functools

import jax
import jax.numpy as jnp
from jax.experimental import pallas as pl
from jax.experimental.pallas import tpu as pltpu


def _pool_kernel(x_ref, p_ref, o_ref, *, hb):
    c = x_ref.shape[2]
    w2 = o_ref.shape[3]
    x = x_ref[0].reshape(hb, 2, c, 2 * w2)
    s = x[:, 0] + x[:, 1]
    s2 = s.reshape(hb * c, 2 * w2)
    p = p_ref[...]
    s_hi = s2.astype(jnp.bfloat16)
    s_lo = (s2 - s_hi.astype(jnp.float32)).astype(jnp.bfloat16)
    y = jax.lax.dot(s_hi, p, preferred_element_type=jnp.float32)
    y = y + jax.lax.dot(s_lo, p, preferred_element_type=jnp.float32)
    o_ref[0] = y.reshape(hb, c, w2)


def kernel(inputs):
    b, h, w, c = inputs.shape
    h2, w2 = h // 2, w // 2

    xt = inputs.transpose(0, 1, 3, 2)
    pair = jnp.repeat(jnp.eye(w2, dtype=jnp.bfloat16) * 0.5, 2, axis=0)

    hb = 64
    while h2 % hb:
        hb //= 2

    out = pl.pallas_call(
        functools.partial(_pool_kernel, hb=hb),
        grid=(b, h2 // hb),
        in_specs=[
            pl.BlockSpec((1, 2 * hb, c, w), lambda bi, hi: (bi, hi, 0, 0)),
            pl.BlockSpec((w, w2), lambda bi, hi: (0, 0)),
        ],
        out_specs=pl.BlockSpec((1, hb, c, w2), lambda bi, hi: (bi, hi, 0, 0)),
        out_shape=jax.ShapeDtypeStruct((b, h2, c, w2), inputs.dtype),
        compiler_params=pltpu.CompilerParams(
            dimension_semantics=(pltpu.PARALLEL, pltpu.ARBITRARY),
        ),
    )(xt, pair)
    return out.transpose(0, 1, 3, 2)


# speedup vs baseline: 1.3187x; 1.0079x over previous
import functools

import jax
import jax.numpy as jnp
from jax.experimental import pallas as pl
from jax.experimental.pallas import tpu as pltpu


def _pool_kernel(x_ref, p_ref, o_ref, *, hb):
    c = x_ref.shape[2]
    w2 = o_ref.shape[3]
    x = x_ref[0].reshape(hb, 2, c, 2 * w2)
    s = x[:, 0] + x[:, 1]
    s2 = s.reshape(hb * c, 2 * w2)
    p = p_ref[...]
    s_hi = s2.astype(jnp.bfloat16)
    s_lo = (s2 - s_hi.astype(jnp.float32)).astype(jnp.bfloat16)
    y = jax.lax.dot(s_hi, p, preferred_element_type=jnp.float32)
    y = y + jax.lax.dot(s_lo, p, preferred_element_type=jnp.float32)
    o_ref[0] = y.reshape(hb, c, w2)


def kernel(inputs):
    b, h, w, c = inputs.shape
    h2, w2 = h // 2, w // 2

    xt = inputs.transpose(0, 1, 3, 2)
    pair = jnp.repeat(jnp.eye(w2, dtype=jnp.bfloat16) * 0.5, 2, axis=0)

    hb = 128
    while h2 % hb:
        hb //= 2

    out = pl.pallas_call(
        functools.partial(_pool_kernel, hb=hb),
        grid=(b, h2 // hb),
        in_specs=[
            pl.BlockSpec((1, 2 * hb, c, w), lambda bi, hi: (bi, hi, 0, 0)),
            pl.BlockSpec((w, w2), lambda bi, hi: (0, 0)),
        ],
        out_specs=pl.BlockSpec((1, hb, c, w2), lambda bi, hi: (bi, hi, 0, 0)),
        out_shape=jax.ShapeDtypeStruct((b, h2, c, w2), inputs.dtype),
        compiler_params=pltpu.CompilerParams(
            dimension_semantics=(pltpu.PARALLEL, pltpu.ARBITRARY),
        ),
    )(xt, pair)
    return out.transpose(0, 1, 3, 2)
